# baseline (speedup 1.0000x reference)
import functools

import jax
import jax.numpy as jnp
from jax import lax
from jax.experimental import pallas as pl
from jax.experimental.pallas import tpu as pltpu
from jax.experimental.pallas import tpu_sc as plsc

BATCH = 16384
EMBED_DIM = 32

_info = plsc.get_sparse_core_info()
NC = _info.num_cores
NS = _info.num_subcores
LANES = _info.num_lanes
NW = NC * NS
B_PER_W = BATCH // NW
CHUNK = 128
NCHUNK = B_PER_W // CHUNK


def _body(uids_hbm, iids_hbm, utab_hbm, itab_hbm, out_hbm,
          uidx_v, iidx_v, urows_v, irows_v, out_v, sem):
    wid = lax.axis_index("s") * NC + lax.axis_index("c")
    base = wid * B_PER_W

    pltpu.sync_copy(uids_hbm.at[pl.ds(base, B_PER_W)], uidx_v)
    pltpu.sync_copy(iids_hbm.at[pl.ds(base, B_PER_W)], iidx_v)

    for c in range(NCHUNK):
        sl = pl.ds(c * CHUNK, CHUNK)
        pltpu.async_copy(utab_hbm.at[uidx_v.at[sl]], urows_v.at[sl, :], sem)
        pltpu.async_copy(itab_hbm.at[iidx_v.at[sl]], irows_v.at[sl, :], sem)
    for c in range(NCHUNK):
        sl = pl.ds(c * CHUNK, CHUNK)
        pltpu.make_async_copy(utab_hbm.at[uidx_v.at[sl]], urows_v.at[sl, :], sem).wait()
        pltpu.make_async_copy(itab_hbm.at[iidx_v.at[sl]], irows_v.at[sl, :], sem).wait()

    lane = lax.iota(jnp.int32, LANES)
    def group(g, carry):
        rows = g * LANES + lane
        acc = jnp.zeros((LANES,), jnp.float32)
        for d in range(EMBED_DIM):
            col = jnp.full((LANES,), d, jnp.int32)
            u = plsc.load_gather(urows_v, [rows, col])
            v = plsc.load_gather(irows_v, [rows, col])
            acc = acc + u * v
        out_v[pl.ds(g * LANES, LANES)] = acc
        return carry

    lax.fori_loop(0, B_PER_W // LANES, group, 0)

    pltpu.sync_copy(out_v, out_hbm.at[pl.ds(base, B_PER_W)])


@functools.partial(
    pl.kernel,
    out_type=jax.ShapeDtypeStruct((BATCH,), jnp.float32),
    mesh=plsc.VectorSubcoreMesh(core_axis_name="c", subcore_axis_name="s"),
    compiler_params=pltpu.CompilerParams(
        use_tc_tiling_on_sc=False, needs_layout_passes=False),
    scratch_types=[
        pltpu.VMEM((B_PER_W,), jnp.int32),
        pltpu.VMEM((B_PER_W,), jnp.int32),
        pltpu.VMEM((B_PER_W, EMBED_DIM), jnp.float32),
        pltpu.VMEM((B_PER_W, EMBED_DIM), jnp.float32),
        pltpu.VMEM((B_PER_W,), jnp.float32),
        pltpu.SemaphoreType.DMA,
    ],
)
def _sc_dot_kernel(uids, iids, utab, itab, out, *scratch):
    _body(uids, iids, utab, itab, out, *scratch)


@jax.jit
def kernel(user_ids, item_ids, user_table, item_table):
    return _sc_dot_kernel(
        user_ids.astype(jnp.int32), item_ids.astype(jnp.int32),
        user_table, item_table)


# speedup vs baseline: 1.0009x; 1.0009x over previous
import functools

import jax
import jax.numpy as jnp
from jax import lax
from jax.experimental import pallas as pl
from jax.experimental.pallas import tpu as pltpu
from jax.experimental.pallas import tpu_sc as plsc

BATCH = 16384
EMBED_DIM = 32
NUM_ROWS = 1000000
PACK = 4
SUP_DIM = PACK * EMBED_DIM

_info = plsc.get_sparse_core_info()
NC = _info.num_cores
NS = _info.num_subcores
LANES = _info.num_lanes
NW = NC * NS
B_PER_W = BATCH // NW
CHUNK = 128
NCHUNK = B_PER_W // CHUNK
NBUF = 3
GROUPS = CHUNK // LANES


def _body(uids_hbm, iids_hbm, utab_hbm, itab_hbm, out_hbm,
          uidx_v, iidx_v, usup_v, isup_v, ubuf_v, ibuf_v, out_v, usem, isem):
    wid = lax.axis_index("s") * NC + lax.axis_index("c")
    base = wid * B_PER_W

    pltpu.sync_copy(uids_hbm.at[pl.ds(base, B_PER_W)], uidx_v)
    pltpu.sync_copy(iids_hbm.at[pl.ds(base, B_PER_W)], iidx_v)

    def prep(j, carry):
        sl = pl.ds(j * LANES, LANES)
        usup_v[sl] = uidx_v[sl] >> 2
        isup_v[sl] = iidx_v[sl] >> 2
        return carry

    lax.fori_loop(0, B_PER_W // LANES, prep, 0)

    def fire(c, buf):
        sl = pl.ds(c * CHUNK, CHUNK)
        pltpu.async_copy(utab_hbm.at[usup_v.at[sl]], ubuf_v.at[buf], usem.at[buf])
        pltpu.async_copy(itab_hbm.at[isup_v.at[sl]], ibuf_v.at[buf], isem.at[buf])

    def drain(c, buf):
        sl = pl.ds(c * CHUNK, CHUNK)
        pltpu.make_async_copy(
            utab_hbm.at[usup_v.at[sl]], ubuf_v.at[buf], usem.at[buf]).wait()
        pltpu.make_async_copy(
            itab_hbm.at[isup_v.at[sl]], ibuf_v.at[buf], isem.at[buf]).wait()

    lane = lax.iota(jnp.int32, LANES)

    for c in range(NBUF):
        fire(c, c)

    for c in range(NCHUNK):
        buf = c % NBUF
        drain(c, buf)
        ub = ubuf_v.at[buf]
        ib = ibuf_v.at[buf]

        def group(g, carry, c=c, ub=ub, ib=ib):
            o = c * CHUNK + g * LANES
            sl = pl.ds(o, LANES)
            ucol = (uidx_v[sl] & 3) * EMBED_DIM
            icol = (iidx_v[sl] & 3) * EMBED_DIM
            slot = g * LANES + lane
            acc0 = jnp.zeros((LANES,), jnp.float32)
            acc1 = jnp.zeros((LANES,), jnp.float32)
            for d in range(EMBED_DIM):
                u = plsc.load_gather(ub, [slot, ucol | d])
                v = plsc.load_gather(ib, [slot, icol | d])
                if d % 2 == 0:
                    acc0 = acc0 + u * v
                else:
                    acc1 = acc1 + u * v
            out_v[sl] = acc0 + acc1
            return carry

        lax.fori_loop(0, GROUPS, group, 0)
        if c + NBUF < NCHUNK:
            fire(c + NBUF, buf)

    pltpu.sync_copy(out_v, out_hbm.at[pl.ds(base, B_PER_W)])


@functools.partial(
    pl.kernel,
    out_type=jax.ShapeDtypeStruct((BATCH,), jnp.float32),
    mesh=plsc.VectorSubcoreMesh(core_axis_name="c", subcore_axis_name="s"),
    compiler_params=pltpu.CompilerParams(needs_layout_passes=False),
    scratch_types=[
        pltpu.VMEM((B_PER_W,), jnp.int32),
        pltpu.VMEM((B_PER_W,), jnp.int32),
        pltpu.VMEM((B_PER_W,), jnp.int32),
        pltpu.VMEM((B_PER_W,), jnp.int32),
        pltpu.VMEM((NBUF, CHUNK, SUP_DIM), jnp.float32),
        pltpu.VMEM((NBUF, CHUNK, SUP_DIM), jnp.float32),
        pltpu.VMEM((B_PER_W,), jnp.float32),
        pltpu.SemaphoreType.DMA((NBUF,)),
        pltpu.SemaphoreType.DMA((NBUF,)),
    ],
)
def _sc_dot_kernel(uids, iids, utab, itab, out, *scratch):
    _body(uids, iids, utab, itab, out, *scratch)


@jax.jit
def kernel(user_ids, item_ids, user_table, item_table):
    utab = user_table.reshape(NUM_ROWS // PACK, SUP_DIM)
    itab = item_table.reshape(NUM_ROWS // PACK, SUP_DIM)
    return _sc_dot_kernel(
        user_ids.astype(jnp.int32), item_ids.astype(jnp.int32), utab, itab)


# speedup vs baseline: 4.7047x; 4.7006x over previous
import functools

import jax
import jax.numpy as jnp
from jax import lax
from jax.experimental import pallas as pl
from jax.experimental.pallas import tpu as pltpu
from jax.experimental.pallas import tpu_sc as plsc

BATCH = 16384
EMBED_DIM = 32
PIECE = 8

_info = plsc.get_sparse_core_info()
NC = _info.num_cores
NS = _info.num_subcores
LANES = _info.num_lanes
NW = NC * NS
B_PER_W = BATCH // NW
NGROUP = B_PER_W // LANES
NBUF = 2
VROWS = NBUF * LANES * EMBED_DIM * PIECE // 128


def _take16(arr, sel):
    return lax.gather(
        arr, sel[:, None],
        lax.GatherDimensionNumbers(
            offset_dims=(), collapsed_slice_dims=(0,), start_index_map=(0,)),
        (1,), mode=lax.GatherScatterMode.PROMISE_IN_BOUNDS)


def _body(uids_hbm, iids_hbm, utab_hbm, itab_hbm, out_hbm,
          uidx_v, iidx_v, uval_v, ival_v, out_v, usem, isem):
    wid = lax.axis_index("s") * NC + lax.axis_index("c")
    base = wid * B_PER_W

    pltpu.sync_copy(uids_hbm.at[pl.ds(base, B_PER_W)], uidx_v)
    pltpu.sync_copy(iids_hbm.at[pl.ds(base, B_PER_W)], iidx_v)

    lane = lax.iota(jnp.int32, LANES)

    def fire(g, buf):
        rs_u = uidx_v[pl.ds(g * LANES, LANES)]
        rs_i = iidx_v[pl.ds(g * LANES, LANES)]

        def one(l, carry):
            sel = jnp.full((LANES,), l, jnp.int32)
            ru = pl.multiple_of(jnp.max(_take16(rs_u, sel)) & -PIECE, PIECE)
            ri = pl.multiple_of(jnp.max(_take16(rs_i, sel)) & -PIECE, PIECE)
            slot = (buf * LANES + l) * EMBED_DIM

            def dchunk(dj, c2):
                for k in range(EMBED_DIM // 2):
                    d = dj * (EMBED_DIM // 2) + k
                    o8 = slot + d
                    row = o8 >> 4
                    col = (o8 & 15) * PIECE
                    pltpu.async_copy(
                        utab_hbm.at[d].at[pl.ds(ru, PIECE)],
                        uval_v.at[row, pl.ds(col, PIECE)], usem.at[buf])
                    pltpu.async_copy(
                        itab_hbm.at[d].at[pl.ds(ri, PIECE)],
                        ival_v.at[row, pl.ds(col, PIECE)], isem.at[buf])
                return c2

            lax.fori_loop(0, 2, dchunk, 0)
            return carry

        lax.fori_loop(0, LANES, one, 0)

    def drain(buf):
        def w(j, carry):
            pltpu.make_async_copy(
                utab_hbm.at[0].at[pl.ds(0, PIECE)],
                uval_v.at[0, pl.ds(0, PIECE)], usem.at[buf]).wait()
            pltpu.make_async_copy(
                itab_hbm.at[0].at[pl.ds(0, PIECE)],
                ival_v.at[0, pl.ds(0, PIECE)], isem.at[buf]).wait()
            return carry

        lax.fori_loop(0, LANES * EMBED_DIM, w, 0)

    def compute(g, buf):
        rs_u = uidx_v[pl.ds(g * LANES, LANES)]
        rs_i = iidx_v[pl.ds(g * LANES, LANES)]
        cu = rs_u & (PIECE - 1)
        ci = rs_i & (PIECE - 1)
        slots = (buf * LANES + lane) * EMBED_DIM
        acc0 = jnp.zeros((LANES,), jnp.float32)
        acc1 = jnp.zeros((LANES,), jnp.float32)
        for d in range(EMBED_DIM):
            o8 = slots + d
            row = o8 >> 4
            colu = ((o8 & 15) << 3) + cu
            coli = ((o8 & 15) << 3) + ci
            u = plsc.load_gather(uval_v, [row, colu])
            v = plsc.load_gather(ival_v, [row, coli])
            if d % 2 == 0:
                acc0 = acc0 + u * v
            else:
                acc1 = acc1 + u * v
        out_v[pl.ds(g * LANES, LANES)] = acc0 + acc1

    fire(0, 0)
    fire(1, 1)

    def step(r, carry):
        g = r * 2
        drain(0)
        compute(g, 0)

        @pl.when(g + 2 < NGROUP)
        def _():
            fire(g + 2, 0)

        drain(1)
        compute(g + 1, 1)

        @pl.when(g + 3 < NGROUP)
        def _():
            fire(g + 3, 1)

        return carry

    lax.fori_loop(0, NGROUP // 2, step, 0)

    pltpu.sync_copy(out_v, out_hbm.at[pl.ds(base, B_PER_W)])


@functools.partial(
    pl.kernel,
    out_type=jax.ShapeDtypeStruct((BATCH,), jnp.float32),
    mesh=plsc.VectorSubcoreMesh(core_axis_name="c", subcore_axis_name="s"),
    compiler_params=pltpu.CompilerParams(needs_layout_passes=False),
    scratch_types=[
        pltpu.VMEM((B_PER_W,), jnp.int32),
        pltpu.VMEM((B_PER_W,), jnp.int32),
        pltpu.VMEM((VROWS, 128), jnp.float32),
        pltpu.VMEM((VROWS, 128), jnp.float32),
        pltpu.VMEM((B_PER_W,), jnp.float32),
        pltpu.SemaphoreType.DMA((NBUF,)),
        pltpu.SemaphoreType.DMA((NBUF,)),
    ],
)
def _sc_dot_kernel(uids, iids, utab_t, itab_t, out, *scratch):
    _body(uids, iids, utab_t, itab_t, out, *scratch)


@jax.jit
def kernel(user_ids, item_ids, user_table, item_table):
    return _sc_dot_kernel(
        user_ids.astype(jnp.int32), item_ids.astype(jnp.int32),
        user_table.T, item_table.T)


# speedup vs baseline: 9.0450x; 1.9225x over previous
import functools

import jax
import jax.numpy as jnp
from jax import lax
from jax.experimental import pallas as pl
from jax.experimental.pallas import tpu as pltpu
from jax.experimental.pallas import tpu_sc as plsc

BATCH = 16384
EMBED_DIM = 32
PIECE = 8

_info = plsc.get_sparse_core_info()
NC = _info.num_cores
NS = _info.num_subcores
LANES = _info.num_lanes
NW = NC * NS
B_PER_W = BATCH // NW
NGROUP = B_PER_W // LANES
NBUF = 2
VROWS = NBUF * LANES * EMBED_DIM * PIECE // 128


def _take16(arr, sel):
    return lax.gather(
        arr, sel[:, None],
        lax.GatherDimensionNumbers(
            offset_dims=(), collapsed_slice_dims=(0,), start_index_map=(0,)),
        (1,), mode=lax.GatherScatterMode.PROMISE_IN_BOUNDS)


def _body(uids_hbm, iids_hbm, utab_hbm, itab_hbm, out_hbm,
          uidx_v, iidx_v, uval_v, ival_v, out_v, usem, isem):
    wid = lax.axis_index("s") * NC + lax.axis_index("c")
    base = wid * B_PER_W

    pltpu.sync_copy(uids_hbm.at[pl.ds(base, B_PER_W)], uidx_v)
    pltpu.sync_copy(iids_hbm.at[pl.ds(base, B_PER_W)], iidx_v)

    lane = lax.iota(jnp.int32, LANES)

    def fire(g, buf):
        rs_u = uidx_v[pl.ds(g * LANES, LANES)]
        rs_i = iidx_v[pl.ds(g * LANES, LANES)]

        def one(l, carry):
            sel = jnp.full((LANES,), l, jnp.int32)
            ru = pl.multiple_of(jnp.max(_take16(rs_u, sel)) & -PIECE, PIECE)
            ri = pl.multiple_of(jnp.max(_take16(rs_i, sel)) & -PIECE, PIECE)
            slot = (buf * LANES + l) * EMBED_DIM

            def dchunk(dj, c2):
                for k in range(EMBED_DIM // 2):
                    d = dj * (EMBED_DIM // 2) + k
                    o8 = slot + d
                    row = o8 >> 4
                    col = (o8 & 15) * PIECE
                    pltpu.async_copy(
                        utab_hbm.at[d].at[pl.ds(ru, PIECE)],
                        uval_v.at[row, pl.ds(col, PIECE)], usem.at[buf])
                    pltpu.async_copy(
                        itab_hbm.at[d].at[pl.ds(ri, PIECE)],
                        ival_v.at[row, pl.ds(col, PIECE)], isem.at[buf])
                return c2

            lax.fori_loop(0, 2, dchunk, 0)
            return carry

        lax.fori_loop(0, LANES, one, 0)

    def drain(buf):
        rows = LANES * EMBED_DIM * PIECE // 128
        pltpu.make_async_copy(
            utab_hbm.at[pl.ds(0, rows), pl.ds(0, 128)],
            uval_v.at[pl.ds(0, rows), :], usem.at[buf]).wait()
        pltpu.make_async_copy(
            itab_hbm.at[pl.ds(0, rows), pl.ds(0, 128)],
            ival_v.at[pl.ds(0, rows), :], isem.at[buf]).wait()

    def compute(g, buf):
        rs_u = uidx_v[pl.ds(g * LANES, LANES)]
        rs_i = iidx_v[pl.ds(g * LANES, LANES)]
        cu = rs_u & (PIECE - 1)
        ci = rs_i & (PIECE - 1)
        slots = (buf * LANES + lane) * EMBED_DIM
        acc0 = jnp.zeros((LANES,), jnp.float32)
        acc1 = jnp.zeros((LANES,), jnp.float32)
        for d in range(EMBED_DIM):
            o8 = slots + d
            row = o8 >> 4
            colu = ((o8 & 15) << 3) + cu
            coli = ((o8 & 15) << 3) + ci
            u = plsc.load_gather(uval_v, [row, colu])
            v = plsc.load_gather(ival_v, [row, coli])
            if d % 2 == 0:
                acc0 = acc0 + u * v
            else:
                acc1 = acc1 + u * v
        out_v[pl.ds(g * LANES, LANES)] = acc0 + acc1

    fire(0, 0)
    fire(1, 1)

    def step(r, carry):
        g = r * 2
        drain(0)
        compute(g, 0)

        @pl.when(g + 2 < NGROUP)
        def _():
            fire(g + 2, 0)

        drain(1)
        compute(g + 1, 1)

        @pl.when(g + 3 < NGROUP)
        def _():
            fire(g + 3, 1)

        return carry

    lax.fori_loop(0, NGROUP // 2, step, 0)

    pltpu.sync_copy(out_v, out_hbm.at[pl.ds(base, B_PER_W)])


@functools.partial(
    pl.kernel,
    out_type=jax.ShapeDtypeStruct((BATCH,), jnp.float32),
    mesh=plsc.VectorSubcoreMesh(core_axis_name="c", subcore_axis_name="s"),
    compiler_params=pltpu.CompilerParams(needs_layout_passes=False),
    scratch_types=[
        pltpu.VMEM((B_PER_W,), jnp.int32),
        pltpu.VMEM((B_PER_W,), jnp.int32),
        pltpu.VMEM((VROWS, 128), jnp.float32),
        pltpu.VMEM((VROWS, 128), jnp.float32),
        pltpu.VMEM((B_PER_W,), jnp.float32),
        pltpu.SemaphoreType.DMA((NBUF,)),
        pltpu.SemaphoreType.DMA((NBUF,)),
    ],
)
def _sc_dot_kernel(uids, iids, utab_t, itab_t, out, *scratch):
    _body(uids, iids, utab_t, itab_t, out, *scratch)


@jax.jit
def kernel(user_ids, item_ids, user_table, item_table):
    return _sc_dot_kernel(
        user_ids.astype(jnp.int32), item_ids.astype(jnp.int32),
        user_table.T, item_table.T)
